# baseline (speedup 1.0000x reference)
import jax
import jax.numpy as jnp
from jax.experimental import pallas as pl
from jax.experimental.pallas import tpu as pltpu


_B_TILE = 512


def _mm_kernel(x_ref, t_ref, o_ref):
    o_ref[...] = jnp.dot(x_ref[...], t_ref[...],
                         preferred_element_type=jnp.float32)


def kernel(inputs, table):
    B, V = inputs.shape
    _, D = table.shape
    return pl.pallas_call(
        _mm_kernel,
        grid=(B // _B_TILE,),
        in_specs=[
            pl.BlockSpec((_B_TILE, V), lambda i: (i, 0)),
            pl.BlockSpec((V, D), lambda i: (0, 0)),
        ],
        out_specs=pl.BlockSpec((_B_TILE, D), lambda i: (i, 0)),
        out_shape=jax.ShapeDtypeStruct((B, D), jnp.float32),
        compiler_params=pltpu.CompilerParams(
            dimension_semantics=("parallel",)),
    )(inputs, table)


# speedup vs baseline: 1.0539x; 1.0539x over previous
import jax
import jax.numpy as jnp
from jax.experimental import pallas as pl
from jax.experimental.pallas import tpu as pltpu


_TILE = 512
_NBUF = 4


def _mm_kernel(x_hbm, t_ref, o_ref, buf, sem):
    ntiles = x_hbm.shape[0] // _TILE

    def copy(i, slot):
        return pltpu.make_async_copy(
            x_hbm.at[pl.ds(i * _TILE, _TILE), :], buf.at[slot], sem.at[slot])

    for s in range(_NBUF):
        copy(s, s).start()

    def body(i, _):
        slot = jax.lax.rem(i, _NBUF)
        copy(i, slot).wait()
        o_ref[pl.ds(i * _TILE, _TILE), :] = jnp.dot(
            buf[slot], t_ref[...], preferred_element_type=jnp.float32)

        @pl.when(i + _NBUF < ntiles)
        def _():
            copy(i + _NBUF, slot).start()

        return 0

    jax.lax.fori_loop(0, ntiles, body, 0)


def kernel(inputs, table):
    B, V = inputs.shape
    _, D = table.shape
    return pl.pallas_call(
        _mm_kernel,
        in_specs=[
            pl.BlockSpec(memory_space=pltpu.MemorySpace.HBM),
            pl.BlockSpec(memory_space=pltpu.MemorySpace.VMEM),
        ],
        out_specs=pl.BlockSpec(memory_space=pltpu.MemorySpace.VMEM),
        out_shape=jax.ShapeDtypeStruct((B, D), jnp.float32),
        scratch_shapes=[
            pltpu.VMEM((_NBUF, _TILE, V), jnp.float32),
            pltpu.SemaphoreType.DMA((_NBUF,)),
        ],
    )(inputs, table)
